# baseline (speedup 1.0000x reference)
import functools

import jax
import jax.numpy as jnp
from jax import lax
from jax.experimental import pallas as pl
from jax.experimental.pallas import tpu as pltpu
from jax.experimental.pallas import tpu_sc as plsc

NC = 2
NS = 16
LANES = 16
K = 128


def _make_edge_scatter(S, Epad, N, D, use_table):
    NW = NC * NS
    NP = -(-N // (NS * 128)) * (NS * 128)
    assert Epad % (NW * K * 2) == 0
    CHT = Epad // (NW * K)
    PAIRS = CHT // 2
    RPT = NP // NS
    ZR = 128
    assert RPT % ZR == 0

    mesh = plsc.VectorSubcoreMesh(
        core_axis_name="c", subcore_axis_name="s",
        num_cores=NC, num_subcores=NS)

    out_type = jax.ShapeDtypeStruct((S, NC, NP, D), jnp.float32)

    def buf_set():
        return [
            pltpu.VMEM((K,), jnp.int32),
            pltpu.VMEM((K,), jnp.int32),
            pltpu.VMEM((K,), jnp.float32),
            pltpu.VMEM((K,), jnp.int32),
            pltpu.VMEM((K,), jnp.int32),
            pltpu.VMEM((K, D), jnp.float32),
            pltpu.VMEM((K,), jnp.float32),
        ] + [pltpu.SemaphoreType.DMA] * 3

    assert ZR == K
    scratch = buf_set() + buf_set() + [
        pltpu.VMEM_SHARED((NP, D), jnp.float32),
    ]

    def body(*refs):
        if use_table:
            rows_h, cols_h, ws_h, table_h, out_h = refs[:5]
            rest = refs[5:]
        else:
            rows_h, cols_h, ws_h, out_h = refs[:4]
            rest = refs[4:]
        A = rest[0:10]
        B = rest[10:20]
        acc = rest[20]
        c = lax.axis_index("c")
        s = lax.axis_index("s")
        wid = s * NC + c

        def run_set(st):
            tile0 = st * Epad
            off = jnp.full((LANES,), st * N, jnp.int32)

            def ebase(t):
                return tile0 + (wid * CHT + t) * K

            def issue_idx(bufs, t):
                rowb, colb, wb, smi = bufs[0], bufs[1], bufs[2], bufs[7]
                b = ebase(t)
                if use_table:
                    pltpu.async_copy(rows_h.at[pl.ds(b, K)], rowb, smi)
                pltpu.async_copy(cols_h.at[pl.ds(b, K)], colb, smi)
                pltpu.async_copy(ws_h.at[pl.ds(b, K)], wb, smi)

            def wait_idx(bufs):
                rowb, colb, wb, smi = bufs[0], bufs[1], bufs[2], bufs[7]
                if use_table:
                    pltpu.make_async_copy(rows_h.at[pl.ds(0, K)], rowb, smi).wait()
                pltpu.make_async_copy(cols_h.at[pl.ds(0, K)], colb, smi).wait()
                pltpu.make_async_copy(ws_h.at[pl.ds(0, K)], wb, smi).wait()

            def compute_indices(bufs):
                rowb, colb, wb = bufs[0], bufs[1], bufs[2]
                gidx, scidx, wsc = bufs[3], bufs[4], bufs[6]
                def gx(j, cy):
                    sl = pl.ds(j * LANES, LANES)
                    if use_table:
                        gidx[sl] = rowb[sl] + off
                    scidx[sl] = colb[sl]
                    wsc[sl] = wb[sl]
                    return cy
                lax.fori_loop(0, K // LANES, gx, 0)

            def issue_gather(bufs):
                gidx, valb, smg = bufs[3], bufs[5], bufs[8]
                pltpu.async_copy(table_h.at[gidx], valb, smg)

            def wait_gather(bufs):
                gidx, valb, smg = bufs[3], bufs[5], bufs[8]
                pltpu.make_async_copy(table_h.at[gidx], valb, smg).wait()

            def issue_scatter(bufs):
                scidx, valb, sms = bufs[4], bufs[5], bufs[9]
                pltpu.async_copy(valb, acc.at[scidx], sms, add=True)

            def wait_scatter(bufs):
                scidx, valb, sms = bufs[4], bufs[5], bufs[9]
                pltpu.make_async_copy(valb, acc.at[scidx], sms).wait()

            def scale(bufs):
                wb, valb = bufs[6], bufs[5]
                def grp(g, cy):
                    wv = wb[pl.ds(g * LANES, LANES)]
                    for i in range(LANES):
                        wsp = jnp.broadcast_to(wv[i], (LANES,))
                        e = g * LANES + i
                        if use_table:
                            for f in range(D // LANES):
                                sl = pl.ds(f * LANES, LANES)
                                valb[e, sl] = valb[e, sl] * wsp
                        else:
                            valb[e, pl.ds(0, LANES)] = wsp
                    return cy
                lax.fori_loop(0, K // LANES, grp, 0)

            zb = A[5]
            def zrow(r, carry):
                for f in range(D // LANES):
                    zb[r, pl.ds(f * LANES, LANES)] = jnp.zeros(
                        (LANES,), jnp.float32)
                return carry
            lax.fori_loop(0, ZR, zrow, 0)
            for z in range(RPT // ZR):
                pltpu.sync_copy(zb, acc.at[pl.ds(s * RPT + z * ZR, ZR)])
            plsc.subcore_barrier()

            if use_table:
                def sync_scatter(bufs):
                    scidx, valb = bufs[4], bufs[5]
                    pltpu.sync_copy(valb, acc.at[scidx], add=True)

                issue_idx(A, 0)
                wait_idx(A)
                compute_indices(A)
                issue_gather(A)
                issue_idx(B, 1)
                issue_idx(A, 2)

                def pair(tp, carry):
                    t = tp * 2
                    wait_gather(A)
                    wait_idx(B)
                    compute_indices(B)
                    issue_gather(B)
                    @pl.when(tp < PAIRS - 1)
                    def _():
                        issue_idx(B, t + 3)
                    scale(A)
                    sync_scatter(A)
                    wait_gather(B)
                    @pl.when(tp < PAIRS - 1)
                    def _():
                        wait_idx(A)
                        compute_indices(A)
                        issue_gather(A)
                        @pl.when(tp < PAIRS - 2)
                        def _():
                            issue_idx(A, t + 4)
                    scale(B)
                    sync_scatter(B)
                    return carry
                lax.fori_loop(0, PAIRS, pair, 0)
            else:
                issue_idx(A, 0)
                issue_idx(B, 1)

                def pair(tp, carry):
                    t = tp * 2
                    for bufs, other, dt in ((A, B, 0), (B, A, 1)):
                        wait_idx(bufs)
                        @pl.when(tp > 0)
                        def _():
                            wait_scatter(bufs)
                        compute_indices(bufs)
                        scale(bufs)
                        issue_scatter(bufs)
                        @pl.when(tp < PAIRS - 1)
                        def _():
                            issue_idx(bufs, t + dt + 2)
                    return carry
                lax.fori_loop(0, PAIRS, pair, 0)
                wait_scatter(A)
                wait_scatter(B)

            plsc.subcore_barrier()
            for z in range(RPT // ZR):
                r0 = s * RPT + z * ZR
                pltpu.sync_copy(acc.at[pl.ds(r0, ZR)],
                                out_h.at[st, c, pl.ds(r0, ZR)])
            plsc.subcore_barrier()

        for st in range(S):
            run_set(st)

    return functools.partial(
        pl.kernel, out_type=out_type, mesh=mesh, scratch_types=scratch)(body)


_R = 1024


def _dinv_tc(degs):
    S, _, n, _ = degs.shape

    def body(d_ref, o_ref):
        d = 1.0 + d_ref[:, 0, :, 0] + d_ref[:, 1, :, 0]
        o_ref[...] = jnp.where(d > 0, lax.rsqrt(jnp.where(d > 0, d, 1.0)), 0.0)

    return pl.pallas_call(
        body,
        out_shape=jax.ShapeDtypeStruct((S, n), jnp.float32),
    )(degs)


def _matmul3_tc(x1, x2, w1, w2, w3, dinv):
    n = x1.shape[0]
    D = w1.shape[1]
    grid = -(-n // _R)

    def body(x1_ref, x2_ref, w1_ref, w2_ref, w3_ref, dv_ref, o_ref):
        i = pl.program_id(0)
        h1 = jnp.dot(x1_ref[...], w1_ref[...],
                     preferred_element_type=jnp.float32)
        h2 = jnp.dot(x1_ref[...], w2_ref[...],
                     preferred_element_type=jnp.float32)
        h3 = jnp.dot(x2_ref[...], w3_ref[...],
                     preferred_element_type=jnp.float32)
        o_ref[0] = h1 * dv_ref[0, pl.ds(i * _R, _R)][:, None]
        o_ref[1] = h2 * dv_ref[1, pl.ds(i * _R, _R)][:, None]
        o_ref[2] = h3 * dv_ref[2, pl.ds(i * _R, _R)][:, None]

    return pl.pallas_call(
        body,
        grid=(grid,),
        in_specs=[
            pl.BlockSpec((_R, x1.shape[1]), lambda i: (i, 0)),
            pl.BlockSpec((_R, x2.shape[1]), lambda i: (i, 0)),
            pl.BlockSpec(w1.shape, lambda i: (0, 0)),
            pl.BlockSpec(w2.shape, lambda i: (0, 0)),
            pl.BlockSpec(w3.shape, lambda i: (0, 0)),
            pl.BlockSpec(dinv.shape, lambda i: (0, 0)),
        ],
        out_specs=pl.BlockSpec((3, _R, D), lambda i: (0, i, 0)),
        out_shape=jax.ShapeDtypeStruct((3, n, D), jnp.float32),
    )(x1, x2, w1, w2, w3, dinv)


def _mid_tc(acc1, h123, dinv, ws, wd, b1, b2, b3):
    _, n, D = h123.shape
    grid = -(-n // _R)

    def body(a_ref, h_ref, dv_ref, ws_ref, wd_ref, b1_ref, b2_ref, b3_ref,
             pro_ref, hw_ref):
        i = pl.program_id(0)
        dv0 = dv_ref[0, pl.ds(i * _R, _R)][:, None]
        dv1 = dv_ref[1, pl.ds(i * _R, _R)][:, None]
        dv2 = dv_ref[2, pl.ds(i * _R, _R)][:, None]
        a0 = a_ref[0, 0] + a_ref[0, 1] + h_ref[0]
        a1 = a_ref[1, 0] + a_ref[1, 1] + h_ref[1]
        a2 = a_ref[2, 0] + a_ref[2, 1] + h_ref[2]
        xs = jax.nn.relu(a0 * dv0 + b1_ref[...][None, :])
        xd = jax.nn.relu(a1 * dv1 + b2_ref[...][None, :])
        pro_ref[...] = a2 * dv2 + b3_ref[...][None, :]
        hs = jnp.dot(xs, ws_ref[...], preferred_element_type=jnp.float32)
        hd = jnp.dot(xd, wd_ref[...], preferred_element_type=jnp.float32)
        hw_ref[0] = hs * dv0
        hw_ref[1] = hd * dv1

    return pl.pallas_call(
        body,
        grid=(grid,),
        in_specs=[
            pl.BlockSpec((3, NC, _R, D), lambda i: (0, 0, i, 0)),
            pl.BlockSpec((3, _R, D), lambda i: (0, i, 0)),
            pl.BlockSpec(dinv.shape, lambda i: (0, 0)),
            pl.BlockSpec(ws.shape, lambda i: (0, 0)),
            pl.BlockSpec(wd.shape, lambda i: (0, 0)),
            pl.BlockSpec(b1.shape, lambda i: (0,)),
            pl.BlockSpec(b2.shape, lambda i: (0,)),
            pl.BlockSpec(b3.shape, lambda i: (0,)),
        ],
        out_specs=[
            pl.BlockSpec((_R, D), lambda i: (i, 0)),
            pl.BlockSpec((2, _R, D), lambda i: (0, i, 0)),
        ],
        out_shape=[
            jax.ShapeDtypeStruct((n, D), jnp.float32),
            jax.ShapeDtypeStruct((2, n, D), jnp.float32),
        ],
    )(acc1, h123, dinv, ws, wd, b1, b2, b3)


def _final_tc(acc2, hw2, dinv, bs, bd, pro, wf1a, wf1b, wf2a, wf2b, bf1, bf2):
    _, n, D = hw2.shape
    grid = -(-n // _R)

    def body(a_ref, h_ref, dv_ref, bs_ref, bd_ref, pro_ref,
             w1a_ref, w1b_ref, w2a_ref, w2b_ref, bf1_ref, bf2_ref,
             xs_ref, xd_ref, f_ref, fp_ref):
        i = pl.program_id(0)
        a0 = a_ref[0, 0] + a_ref[0, 1] + h_ref[0]
        a1 = a_ref[1, 0] + a_ref[1, 1] + h_ref[1]
        x_sim = a0 * dv_ref[0, pl.ds(i * _R, _R)][:, None] + bs_ref[...][None, :]
        x_dist = a1 * dv_ref[1, pl.ds(i * _R, _R)][:, None] + bd_ref[...][None, :]
        fused = (jnp.dot(x_sim, w1a_ref[...], preferred_element_type=jnp.float32)
                 + jnp.dot(x_dist, w1b_ref[...], preferred_element_type=jnp.float32)
                 + bf1_ref[...][None, :])
        fp = (jnp.dot(fused, w2a_ref[...], preferred_element_type=jnp.float32)
              + jnp.dot(pro_ref[...], w2b_ref[...], preferred_element_type=jnp.float32)
              + bf2_ref[...][None, :])
        xs_ref[...] = x_sim
        xd_ref[...] = x_dist
        f_ref[...] = fused
        fp_ref[...] = fp

    os = jax.ShapeDtypeStruct((n, D), jnp.float32)
    return pl.pallas_call(
        body,
        grid=(grid,),
        in_specs=[
            pl.BlockSpec((2, NC, _R, D), lambda i: (0, 0, i, 0)),
            pl.BlockSpec((2, _R, D), lambda i: (0, i, 0)),
            pl.BlockSpec(dinv.shape, lambda i: (0, 0)),
            pl.BlockSpec(bs.shape, lambda i: (0,)),
            pl.BlockSpec(bd.shape, lambda i: (0,)),
            pl.BlockSpec((_R, D), lambda i: (i, 0)),
            pl.BlockSpec(wf1a.shape, lambda i: (0, 0)),
            pl.BlockSpec(wf1b.shape, lambda i: (0, 0)),
            pl.BlockSpec(wf2a.shape, lambda i: (0, 0)),
            pl.BlockSpec(wf2b.shape, lambda i: (0, 0)),
            pl.BlockSpec(bf1.shape, lambda i: (0,)),
            pl.BlockSpec(bf2.shape, lambda i: (0,)),
        ],
        out_specs=[pl.BlockSpec((_R, D), lambda i: (i, 0))] * 4,
        out_shape=[os, os, os, os],
    )(acc2, hw2, dinv, bs, bd, pro, wf1a, wf1b, wf2a, wf2b, bf1, bf2)


def kernel(x_RNA, x_ADT, sim_edge_index, sim_edge_weight, dist_edge_index,
           dist_edge_weight, common_edge_index, common_edge_weight,
           W1, b1, W2, b2, W3, b3, Ws, bs, Wd, bd, Wf1, bf1, Wf2, bf2):
    n, D = x_RNA.shape[0], W1.shape[1]
    E = sim_edge_weight.shape[0]

    rows3 = jnp.stack([sim_edge_index[0], dist_edge_index[0],
                       common_edge_index[0]])
    cols3 = jnp.stack([sim_edge_index[1], dist_edge_index[1],
                       common_edge_index[1]])
    ws3 = jnp.stack([sim_edge_weight, dist_edge_weight, common_edge_weight])

    NW = NC * NS
    cht = -(-E // (K * NW))
    cht += cht % 2
    epad = cht * K * NW
    pad = ((0, 0), (0, epad - E))
    rows3f = jnp.pad(rows3, pad).reshape(-1)
    cols3f = jnp.pad(cols3, pad).reshape(-1)
    ws3f = jnp.pad(ws3, pad).reshape(-1)

    deg_fn = _make_edge_scatter(3, epad, n, D, use_table=False)
    degs = deg_fn(rows3f, cols3f, ws3f)
    dinv = _dinv_tc(degs)

    h123 = _matmul3_tc(x_RNA, x_ADT, W1, W2, W3, dinv)

    prop3_fn = _make_edge_scatter(3, epad, n, D, use_table=True)
    acc1 = prop3_fn(rows3f, cols3f, ws3f, h123.reshape(3 * n, D))

    pro, hw2 = _mid_tc(acc1, h123, dinv, Ws, Wd, b1, b2, b3)

    prop2_fn = _make_edge_scatter(2, epad, n, D, use_table=True)
    acc2 = prop2_fn(rows3f[:2 * epad], cols3f[:2 * epad], ws3f[:2 * epad],
                    hw2.reshape(2 * n, D))

    x_sim, x_dist, fused, fused_pro = _final_tc(
        acc2, hw2, dinv, bs, bd, pro,
        Wf1[:D], Wf1[D:], Wf2[:D], Wf2[D:], bf1, bf2)

    return (x_sim, x_dist, fused, fused_pro, pro)


# speedup vs baseline: 2.6675x; 2.6675x over previous
import functools

import jax
import jax.numpy as jnp
from jax import lax
from jax.experimental import pallas as pl
from jax.experimental.pallas import tpu as pltpu
from jax.experimental.pallas import tpu_sc as plsc

NC = 2
NS = 16
LANES = 16
K = 128


def _make_edge_scatter(S, Epad, N, D, use_table):
    NW = NC * NS
    NP = -(-N // (NS * 128)) * (NS * 128)
    assert Epad % (NW * K * 2) == 0
    CHT = Epad // (NW * K)
    PAIRS = CHT // 2
    RPT = NP // NS
    ZR = 128
    assert RPT % ZR == 0

    mesh = plsc.VectorSubcoreMesh(
        core_axis_name="c", subcore_axis_name="s",
        num_cores=NC, num_subcores=NS)

    out_type = jax.ShapeDtypeStruct((S, NC, NP, D), jnp.float32)

    def buf_set():
        return [
            pltpu.VMEM((K,), jnp.int32),
            pltpu.VMEM((K,), jnp.int32),
            pltpu.VMEM((K,), jnp.float32),
            pltpu.VMEM((K,), jnp.int32),
            pltpu.VMEM((K,), jnp.int32),
            pltpu.VMEM((K, D), jnp.float32),
            pltpu.VMEM((K,), jnp.float32),
        ] + [pltpu.SemaphoreType.DMA] * 3

    assert ZR == K
    scratch = buf_set() + buf_set() + [
        pltpu.VMEM_SHARED((NP, D), jnp.float32),
    ]

    def body(*refs):
        if use_table:
            rows_h, cols_h, ws_h, table_h, out_h = refs[:5]
            rest = refs[5:]
        else:
            rows_h, cols_h, ws_h, out_h = refs[:4]
            rest = refs[4:]
        A = rest[0:10]
        B = rest[10:20]
        acc = rest[20]
        c = lax.axis_index("c")
        s = lax.axis_index("s")
        wid = s * NC + c

        def run_set(st):
            tile0 = st * Epad
            off = jnp.full((LANES,), st * N, jnp.int32)

            def ebase(t):
                return tile0 + (wid * CHT + t) * K

            def issue_idx(bufs, t):
                rowb, colb, wb, smi = bufs[0], bufs[1], bufs[2], bufs[7]
                b = ebase(t)
                if use_table:
                    pltpu.async_copy(rows_h.at[pl.ds(b, K)], rowb, smi)
                pltpu.async_copy(cols_h.at[pl.ds(b, K)], colb, smi)
                pltpu.async_copy(ws_h.at[pl.ds(b, K)], wb, smi)

            def wait_idx(bufs):
                rowb, colb, wb, smi = bufs[0], bufs[1], bufs[2], bufs[7]
                if use_table:
                    pltpu.make_async_copy(rows_h.at[pl.ds(0, K)], rowb, smi).wait()
                pltpu.make_async_copy(cols_h.at[pl.ds(0, K)], colb, smi).wait()
                pltpu.make_async_copy(ws_h.at[pl.ds(0, K)], wb, smi).wait()

            def compute_indices(bufs):
                rowb, colb, wb = bufs[0], bufs[1], bufs[2]
                gidx, scidx, wsc = bufs[3], bufs[4], bufs[6]
                def gx(j, cy):
                    sl = pl.ds(j * LANES, LANES)
                    if use_table:
                        gidx[sl] = rowb[sl] + off
                    scidx[sl] = colb[sl]
                    wsc[sl] = wb[sl]
                    return cy
                lax.fori_loop(0, K // LANES, gx, 0)

            def issue_gather(bufs):
                gidx, valb, smg = bufs[3], bufs[5], bufs[8]
                pltpu.async_copy(table_h.at[gidx], valb, smg)

            def wait_gather(bufs):
                gidx, valb, smg = bufs[3], bufs[5], bufs[8]
                pltpu.make_async_copy(table_h.at[gidx], valb, smg).wait()

            def issue_scatter(bufs):
                scidx, valb, sms = bufs[4], bufs[5], bufs[9]
                pltpu.async_copy(valb, acc.at[scidx], sms, add=True)

            def wait_scatter(bufs):
                scidx, valb, sms = bufs[4], bufs[5], bufs[9]
                pltpu.make_async_copy(valb, acc.at[scidx], sms).wait()

            def scale(bufs):
                wb, valb = bufs[6], bufs[5]
                def grp(g, cy):
                    wv = wb[pl.ds(g * LANES, LANES)]
                    for i in range(LANES):
                        wsp = jnp.broadcast_to(wv[i], (LANES,))
                        e = g * LANES + i
                        if use_table:
                            for f in range(D // LANES):
                                sl = pl.ds(f * LANES, LANES)
                                valb[e, sl] = valb[e, sl] * wsp
                        else:
                            valb[e, pl.ds(0, LANES)] = wsp
                    return cy
                lax.fori_loop(0, K // LANES, grp, 0)

            zb = A[5]
            def zrow(r, carry):
                for f in range(D // LANES):
                    zb[r, pl.ds(f * LANES, LANES)] = jnp.zeros(
                        (LANES,), jnp.float32)
                return carry
            lax.fori_loop(0, ZR, zrow, 0)
            for z in range(RPT // ZR):
                pltpu.sync_copy(zb, acc.at[pl.ds(s * RPT + z * ZR, ZR)])
            plsc.subcore_barrier()

            if use_table:
                def sync_scatter(bufs):
                    scidx, valb = bufs[4], bufs[5]
                    pltpu.sync_copy(valb, acc.at[scidx], add=True)

                issue_idx(A, 0)
                wait_idx(A)
                compute_indices(A)
                issue_gather(A)
                issue_idx(B, 1)
                issue_idx(A, 2)

                def pair(tp, carry):
                    t = tp * 2
                    wait_idx(B)
                    compute_indices(B)
                    issue_gather(B)
                    @pl.when(tp < PAIRS - 1)
                    def _():
                        issue_idx(B, t + 3)
                    wait_gather(A)
                    scale(A)
                    sync_scatter(A)
                    @pl.when(tp < PAIRS - 1)
                    def _():
                        wait_idx(A)
                        compute_indices(A)
                        issue_gather(A)
                        @pl.when(tp < PAIRS - 2)
                        def _():
                            issue_idx(A, t + 4)
                    wait_gather(B)
                    scale(B)
                    sync_scatter(B)
                    return carry
                lax.fori_loop(0, PAIRS, pair, 0)
            else:
                issue_idx(A, 0)
                issue_idx(B, 1)

                def pair(tp, carry):
                    t = tp * 2
                    for bufs, other, dt in ((A, B, 0), (B, A, 1)):
                        wait_idx(bufs)
                        @pl.when(tp > 0)
                        def _():
                            wait_scatter(bufs)
                        compute_indices(bufs)
                        scale(bufs)
                        issue_scatter(bufs)
                        @pl.when(tp < PAIRS - 1)
                        def _():
                            issue_idx(bufs, t + dt + 2)
                    return carry
                lax.fori_loop(0, PAIRS, pair, 0)
                wait_scatter(A)
                wait_scatter(B)

            plsc.subcore_barrier()
            for z in range(RPT // ZR):
                r0 = s * RPT + z * ZR
                pltpu.sync_copy(acc.at[pl.ds(r0, ZR)],
                                out_h.at[st, c, pl.ds(r0, ZR)])
            plsc.subcore_barrier()

        for st in range(S):
            run_set(st)

    return functools.partial(
        pl.kernel, out_type=out_type, mesh=mesh, scratch_types=scratch)(body)


_R = 1024


def _dinv_tc(degs):
    S, _, n, _ = degs.shape

    def body(d_ref, o_ref):
        d = 1.0 + d_ref[:, 0, :, 0] + d_ref[:, 1, :, 0]
        o_ref[...] = jnp.where(d > 0, lax.rsqrt(jnp.where(d > 0, d, 1.0)), 0.0)

    return pl.pallas_call(
        body,
        out_shape=jax.ShapeDtypeStruct((S, n), jnp.float32),
    )(degs)


def _matmul3_tc(x1, x2, w1, w2, w3, dinv):
    n = x1.shape[0]
    D = w1.shape[1]
    grid = -(-n // _R)

    def body(x1_ref, x2_ref, w1_ref, w2_ref, w3_ref, dv_ref, o_ref):
        i = pl.program_id(0)
        h1 = jnp.dot(x1_ref[...], w1_ref[...],
                     preferred_element_type=jnp.float32)
        h2 = jnp.dot(x1_ref[...], w2_ref[...],
                     preferred_element_type=jnp.float32)
        h3 = jnp.dot(x2_ref[...], w3_ref[...],
                     preferred_element_type=jnp.float32)
        o_ref[0] = h1 * dv_ref[0, pl.ds(i * _R, _R)][:, None]
        o_ref[1] = h2 * dv_ref[1, pl.ds(i * _R, _R)][:, None]
        o_ref[2] = h3 * dv_ref[2, pl.ds(i * _R, _R)][:, None]

    return pl.pallas_call(
        body,
        grid=(grid,),
        in_specs=[
            pl.BlockSpec((_R, x1.shape[1]), lambda i: (i, 0)),
            pl.BlockSpec((_R, x2.shape[1]), lambda i: (i, 0)),
            pl.BlockSpec(w1.shape, lambda i: (0, 0)),
            pl.BlockSpec(w2.shape, lambda i: (0, 0)),
            pl.BlockSpec(w3.shape, lambda i: (0, 0)),
            pl.BlockSpec(dinv.shape, lambda i: (0, 0)),
        ],
        out_specs=pl.BlockSpec((3, _R, D), lambda i: (0, i, 0)),
        out_shape=jax.ShapeDtypeStruct((3, n, D), jnp.float32),
    )(x1, x2, w1, w2, w3, dinv)


def _mid_tc(acc1, h123, dinv, ws, wd, b1, b2, b3):
    _, n, D = h123.shape
    grid = -(-n // _R)

    def body(a_ref, h_ref, dv_ref, ws_ref, wd_ref, b1_ref, b2_ref, b3_ref,
             pro_ref, hw_ref):
        i = pl.program_id(0)
        dv0 = dv_ref[0, pl.ds(i * _R, _R)][:, None]
        dv1 = dv_ref[1, pl.ds(i * _R, _R)][:, None]
        dv2 = dv_ref[2, pl.ds(i * _R, _R)][:, None]
        a0 = a_ref[0, 0] + a_ref[0, 1] + h_ref[0]
        a1 = a_ref[1, 0] + a_ref[1, 1] + h_ref[1]
        a2 = a_ref[2, 0] + a_ref[2, 1] + h_ref[2]
        xs = jax.nn.relu(a0 * dv0 + b1_ref[...][None, :])
        xd = jax.nn.relu(a1 * dv1 + b2_ref[...][None, :])
        pro_ref[...] = a2 * dv2 + b3_ref[...][None, :]
        hs = jnp.dot(xs, ws_ref[...], preferred_element_type=jnp.float32)
        hd = jnp.dot(xd, wd_ref[...], preferred_element_type=jnp.float32)
        hw_ref[0] = hs * dv0
        hw_ref[1] = hd * dv1

    return pl.pallas_call(
        body,
        grid=(grid,),
        in_specs=[
            pl.BlockSpec((3, NC, _R, D), lambda i: (0, 0, i, 0)),
            pl.BlockSpec((3, _R, D), lambda i: (0, i, 0)),
            pl.BlockSpec(dinv.shape, lambda i: (0, 0)),
            pl.BlockSpec(ws.shape, lambda i: (0, 0)),
            pl.BlockSpec(wd.shape, lambda i: (0, 0)),
            pl.BlockSpec(b1.shape, lambda i: (0,)),
            pl.BlockSpec(b2.shape, lambda i: (0,)),
            pl.BlockSpec(b3.shape, lambda i: (0,)),
        ],
        out_specs=[
            pl.BlockSpec((_R, D), lambda i: (i, 0)),
            pl.BlockSpec((2, _R, D), lambda i: (0, i, 0)),
        ],
        out_shape=[
            jax.ShapeDtypeStruct((n, D), jnp.float32),
            jax.ShapeDtypeStruct((2, n, D), jnp.float32),
        ],
    )(acc1, h123, dinv, ws, wd, b1, b2, b3)


def _final_tc(acc2, hw2, dinv, bs, bd, pro, wf1a, wf1b, wf2a, wf2b, bf1, bf2):
    _, n, D = hw2.shape
    grid = -(-n // _R)

    def body(a_ref, h_ref, dv_ref, bs_ref, bd_ref, pro_ref,
             w1a_ref, w1b_ref, w2a_ref, w2b_ref, bf1_ref, bf2_ref,
             xs_ref, xd_ref, f_ref, fp_ref):
        i = pl.program_id(0)
        a0 = a_ref[0, 0] + a_ref[0, 1] + h_ref[0]
        a1 = a_ref[1, 0] + a_ref[1, 1] + h_ref[1]
        x_sim = a0 * dv_ref[0, pl.ds(i * _R, _R)][:, None] + bs_ref[...][None, :]
        x_dist = a1 * dv_ref[1, pl.ds(i * _R, _R)][:, None] + bd_ref[...][None, :]
        fused = (jnp.dot(x_sim, w1a_ref[...], preferred_element_type=jnp.float32)
                 + jnp.dot(x_dist, w1b_ref[...], preferred_element_type=jnp.float32)
                 + bf1_ref[...][None, :])
        fp = (jnp.dot(fused, w2a_ref[...], preferred_element_type=jnp.float32)
              + jnp.dot(pro_ref[...], w2b_ref[...], preferred_element_type=jnp.float32)
              + bf2_ref[...][None, :])
        xs_ref[...] = x_sim
        xd_ref[...] = x_dist
        f_ref[...] = fused
        fp_ref[...] = fp

    os = jax.ShapeDtypeStruct((n, D), jnp.float32)
    return pl.pallas_call(
        body,
        grid=(grid,),
        in_specs=[
            pl.BlockSpec((2, NC, _R, D), lambda i: (0, 0, i, 0)),
            pl.BlockSpec((2, _R, D), lambda i: (0, i, 0)),
            pl.BlockSpec(dinv.shape, lambda i: (0, 0)),
            pl.BlockSpec(bs.shape, lambda i: (0,)),
            pl.BlockSpec(bd.shape, lambda i: (0,)),
            pl.BlockSpec((_R, D), lambda i: (i, 0)),
            pl.BlockSpec(wf1a.shape, lambda i: (0, 0)),
            pl.BlockSpec(wf1b.shape, lambda i: (0, 0)),
            pl.BlockSpec(wf2a.shape, lambda i: (0, 0)),
            pl.BlockSpec(wf2b.shape, lambda i: (0, 0)),
            pl.BlockSpec(bf1.shape, lambda i: (0,)),
            pl.BlockSpec(bf2.shape, lambda i: (0,)),
        ],
        out_specs=[pl.BlockSpec((_R, D), lambda i: (i, 0))] * 4,
        out_shape=[os, os, os, os],
    )(acc2, hw2, dinv, bs, bd, pro, wf1a, wf1b, wf2a, wf2b, bf1, bf2)


def kernel(x_RNA, x_ADT, sim_edge_index, sim_edge_weight, dist_edge_index,
           dist_edge_weight, common_edge_index, common_edge_weight,
           W1, b1, W2, b2, W3, b3, Ws, bs, Wd, bd, Wf1, bf1, Wf2, bf2):
    n, D = x_RNA.shape[0], W1.shape[1]
    E = sim_edge_weight.shape[0]

    rows3 = jnp.stack([sim_edge_index[0], dist_edge_index[0],
                       common_edge_index[0]])
    cols3 = jnp.stack([sim_edge_index[1], dist_edge_index[1],
                       common_edge_index[1]])
    ws3 = jnp.stack([sim_edge_weight, dist_edge_weight, common_edge_weight])

    NW = NC * NS
    cht = -(-E // (K * NW))
    cht += cht % 2
    epad = cht * K * NW
    pad_len = epad - E
    pidx = jnp.arange(pad_len, dtype=jnp.int32) % n
    pad2 = jnp.broadcast_to(pidx, (3, pad_len))
    rows3f = jnp.concatenate([rows3, pad2], axis=1).reshape(-1)
    cols3f = jnp.concatenate([cols3, pad2], axis=1).reshape(-1)
    ws3f = jnp.concatenate(
        [ws3, jnp.zeros((3, pad_len), jnp.float32)], axis=1).reshape(-1)

    deg_fn = _make_edge_scatter(3, epad, n, D, use_table=False)
    degs = deg_fn(rows3f, cols3f, ws3f)
    dinv = _dinv_tc(degs)

    h123 = _matmul3_tc(x_RNA, x_ADT, W1, W2, W3, dinv)

    prop3_fn = _make_edge_scatter(3, epad, n, D, use_table=True)
    acc1 = prop3_fn(rows3f, cols3f, ws3f, h123.reshape(3 * n, D))

    pro, hw2 = _mid_tc(acc1, h123, dinv, Ws, Wd, b1, b2, b3)

    prop2_fn = _make_edge_scatter(2, epad, n, D, use_table=True)
    acc2 = prop2_fn(rows3f[:2 * epad], cols3f[:2 * epad], ws3f[:2 * epad],
                    hw2.reshape(2 * n, D))

    x_sim, x_dist, fused, fused_pro = _final_tc(
        acc2, hw2, dinv, bs, bd, pro,
        Wf1[:D], Wf1[D:], Wf2[:D], Wf2[D:], bf1, bf2)

    return (x_sim, x_dist, fused, fused_pro, pro)
